# Initial kernel scaffold; baseline (speedup 1.0000x reference)
import jax
import jax.numpy as jnp
from jax.experimental import pallas as pl


def kernel(x, table):
    raise NotImplementedError("write your pallas kernel here")


# speedup vs baseline: 4.7467x; 4.7467x over previous
import functools

import jax
import jax.numpy as jnp
from jax import lax
from jax.experimental import pallas as pl
from jax.experimental.pallas import tpu as pltpu
from jax.experimental.pallas import tpu_sc as plsc

_info = plsc.get_sparse_core_info()
_NC, _NS, _L = _info.num_cores, _info.num_subcores, _info.num_lanes
_NW = _NC * _NS

_IDX_W = 128
_GATHERS = 8
_CHUNK = _IDX_W * _GATHERS


@functools.cache
def _build(V, D, B):
    assert B % (_NW * _CHUNK) == 0, (V, D, B)
    rows_per_w = B // _NW
    chunks = B // (_NW * _CHUNK)
    mesh = plsc.VectorSubcoreMesh(core_axis_name="c", subcore_axis_name="s")

    @functools.partial(
        pl.kernel,
        mesh=mesh,
        out_type=jax.ShapeDtypeStruct((B, D), jnp.float32),
        scratch_types=[
            pltpu.VMEM((_GATHERS, _IDX_W), jnp.int32),
            pltpu.VMEM((_CHUNK, D), jnp.float32),
            pltpu.SemaphoreType.DMA,
        ],
        compiler_params=pltpu.CompilerParams(use_tc_tiling_on_sc=False),
    )
    def emb(table_hbm, idx_hbm, out_hbm, idx_v, rows_v, sem):
        wid = lax.axis_index("s") * _NC + lax.axis_index("c")
        idx_row0 = wid * (rows_per_w // _IDX_W)

        def chunk_body(i, carry):
            r0 = idx_row0 + i * _GATHERS
            pltpu.sync_copy(idx_hbm.at[pl.ds(r0, _GATHERS)], idx_v)
            copies = [
                pltpu.async_copy(
                    table_hbm.at[idx_v.at[j]],
                    rows_v.at[pl.ds(j * _IDX_W, _IDX_W)],
                    sem,
                )
                for j in range(_GATHERS)
            ]
            for c in copies:
                c.wait()
            pltpu.sync_copy(rows_v, out_hbm.at[pl.ds(r0 * _IDX_W, _CHUNK)])
            return carry

        lax.fori_loop(0, chunks, chunk_body, 0)

    return emb


def kernel(x, table):
    V, D = table.shape
    B = x.size
    idx = x.reshape(B // _IDX_W, _IDX_W).astype(jnp.int32)
    out = _build(V, D, B)(table, idx)
    return out.reshape(*x.shape, D)
